# baseline (speedup 1.0000x reference)
import jax
import jax.numpy as jnp
from jax import lax
from jax.experimental import pallas as pl
from jax.experimental.pallas import tpu as pltpu
from jax.experimental.pallas import tpu_sc as plsc

EMBED = 64
GROUP = 16
NC, NS = 2, 16
NW = NC * NS


def _make_gather(n_sent: int, s_len: int):
    sent_per_w = n_sent // NW
    n_groups = sent_per_w // GROUP
    assert n_sent % NW == 0 and sent_per_w % GROUP == 0 and n_groups % 2 == 0
    mesh = plsc.VectorSubcoreMesh(core_axis_name="c", subcore_axis_name="s",
                                  num_cores=NC, num_subcores=NS)

    def body(idx_hbm, tab, out_hbm, idx_v, buf0, buf1, sem):
        wid = lax.axis_index("s") * NC + lax.axis_index("c")
        base = wid * sent_per_w
        bufs = (buf0, buf1)

        pltpu.sync_copy(idx_hbm.at[pl.ds(base, sent_per_w)], idx_v)

        def fire(g, buf):
            for j in range(GROUP):
                pltpu.async_copy(
                    tab.at[idx_v.at[g * GROUP + j]],
                    buf.at[pl.ds(j * s_len, s_len)], sem)

        def drain(buf):
            pltpu.make_async_copy(
                tab.at[pl.ds(0, GROUP * s_len)], buf, sem).wait()

        fire(0, bufs[0])

        @pl.loop(0, n_groups, step=2)
        def _(g):
            for b in range(2):
                gg = g + b
                drain(bufs[b])

                @pl.when(gg + 1 < n_groups)
                def _():
                    fire(gg + 1, bufs[1 - b])

                pltpu.sync_copy(
                    bufs[b],
                    out_hbm.at[pl.ds((base + gg * GROUP) * s_len,
                                     GROUP * s_len)])

    out_sd = jax.ShapeDtypeStruct((n_sent * s_len, EMBED), jnp.float32)
    return pl.kernel(
        body,
        out_type=out_sd,
        mesh=mesh,
        scratch_types=[
            pltpu.VMEM((sent_per_w, s_len), jnp.int32),
            pltpu.VMEM((GROUP * s_len, EMBED), jnp.float32),
            pltpu.VMEM((GROUP * s_len, EMBED), jnp.float32),
            pltpu.SemaphoreType.DMA,
        ],
        compiler_params=pltpu.CompilerParams(use_tc_tiling_on_sc=False),
    )


def kernel(src_indices, tgt_indices, src_table, tgt_table):
    b, s = src_indices.shape
    lookup = _make_gather(b, s)
    out_src = lookup(src_indices.astype(jnp.int32), src_table)
    out_tgt = lookup(tgt_indices.astype(jnp.int32), tgt_table)
    return (out_src.reshape(b, s, EMBED), out_tgt.reshape(b, s, EMBED))


# speedup vs baseline: 1.0045x; 1.0045x over previous
import jax
import jax.numpy as jnp
from jax import lax
from jax.experimental import pallas as pl
from jax.experimental.pallas import tpu as pltpu
from jax.experimental.pallas import tpu_sc as plsc

EMBED = 64
GROUP = 8
NC, NS = 2, 16
NW = NC * NS


def _make_gather(n_sent: int, s_len: int):
    sent_per_w = n_sent // NW
    n_groups = sent_per_w // GROUP
    assert n_sent % NW == 0 and sent_per_w % GROUP == 0 and n_groups % 2 == 0
    mesh = plsc.VectorSubcoreMesh(core_axis_name="c", subcore_axis_name="s",
                                  num_cores=NC, num_subcores=NS)

    def body(idx_hbm, tab, out_hbm, idx_v, buf0, buf1, sem):
        wid = lax.axis_index("s") * NC + lax.axis_index("c")
        base = wid * sent_per_w
        bufs = (buf0, buf1)

        pltpu.sync_copy(idx_hbm.at[pl.ds(base, sent_per_w)], idx_v)

        def fire(g, buf):
            for j in range(GROUP):
                pltpu.async_copy(
                    tab.at[idx_v.at[g * GROUP + j]],
                    buf.at[pl.ds(j * s_len, s_len)], sem)

        def drain(buf):
            pltpu.make_async_copy(
                tab.at[pl.ds(0, GROUP * s_len)], buf, sem).wait()

        fire(0, bufs[0])

        @pl.loop(0, n_groups, step=2)
        def _(g):
            for b in range(2):
                gg = g + b
                drain(bufs[b])

                @pl.when(gg + 1 < n_groups)
                def _():
                    fire(gg + 1, bufs[1 - b])

                pltpu.sync_copy(
                    bufs[b],
                    out_hbm.at[pl.ds((base + gg * GROUP) * s_len,
                                     GROUP * s_len)])

    out_sd = jax.ShapeDtypeStruct((n_sent * s_len, EMBED), jnp.float32)
    return pl.kernel(
        body,
        out_type=out_sd,
        mesh=mesh,
        scratch_types=[
            pltpu.VMEM((sent_per_w, s_len), jnp.int32),
            pltpu.VMEM((GROUP * s_len, EMBED), jnp.float32),
            pltpu.VMEM((GROUP * s_len, EMBED), jnp.float32),
            pltpu.SemaphoreType.DMA,
        ],
        compiler_params=pltpu.CompilerParams(use_tc_tiling_on_sc=False),
    )


def kernel(src_indices, tgt_indices, src_table, tgt_table):
    b, s = src_indices.shape
    lookup = _make_gather(b, s)
    out_src = lookup(src_indices.astype(jnp.int32), src_table)
    out_tgt = lookup(tgt_indices.astype(jnp.int32), tgt_table)
    return (out_src.reshape(b, s, EMBED), out_tgt.reshape(b, s, EMBED))
